# baseline (speedup 1.0000x reference)
import functools

import jax
import jax.numpy as jnp
from jax import lax
from jax.experimental import pallas as pl
from jax.experimental.pallas import tpu as pltpu
from jax.experimental.pallas import tpu_sc as plsc

_DEPTH = 1000
_B, _T = 1024, 200
_NW = 32
_BPW = _B // _NW
_CH = 40
_CPB = _T // _CH
_NCHUNK = _BPW * _CPB
_RPW = _BPW * _T
_LANE = 16
_NGRP = (_CH + _LANE - 1) // _LANE


def _set_vals(buf, xv, off, val):
    lanes = lax.broadcasted_iota(jnp.int32, (_LANE,), 0)
    vals = jnp.full((_LANE,), val, jnp.float32)
    for j in range(_NGRP):
        xm = xv[pl.ds(off + j * _LANE, _LANE)] % _DEPTH
        rows = lanes + (j * _LANE)
        nvalid = _CH - j * _LANE
        if nvalid >= _LANE:
            plsc.store_scatter(buf, [rows, xm], vals)
        else:
            plsc.store_scatter(buf, [rows, xm], vals, mask=lanes < nvalid)


def _sc_body(x_hbm, out_hbm, xv, b0, b1, s0, s1):
    wid = lax.axis_index("s") * 2 + lax.axis_index("c")
    bbase = wid * _BPW
    pltpu.sync_copy(x_hbm.at[pl.ds(wid * _RPW, _RPW)], xv.at[pl.ds(0, _RPW)])

    zero16 = jnp.zeros((_LANE,), jnp.float32)
    for buf in (b0, b1):
        def _zrow(r, _, buf=buf):
            for c in range(_DEPTH // _LANE):
                buf[r, pl.ds(c * _LANE, _LANE)] = zero16
            buf[r, pl.ds(_DEPTH - _LANE, _LANE)] = zero16
            return 0
        lax.fori_loop(0, _CH, _zrow, 0)

    bufs, sems = (b0, b1), (s0, s1)

    def _dst(c):
        return out_hbm.at[bbase + c // _CPB, pl.ds((c % _CPB) * _CH, _CH)]

    for b in range(2):
        _set_vals(bufs[b], xv, b * _CH, 1.0)
        pltpu.async_copy(bufs[b], _dst(b), sems[b])

    def _ring(g, _):
        for b in range(2):
            c = 2 * g + b
            pltpu.make_async_copy(bufs[b], _dst(c), sems[b]).wait()
            _set_vals(bufs[b], xv, (c - 2) * _CH, 0.0)
            _set_vals(bufs[b], xv, c * _CH, 1.0)
            pltpu.async_copy(bufs[b], _dst(c), sems[b])
        return 0

    lax.fori_loop(1, _NCHUNK // 2, _ring, 0)

    for b in range(2):
        pltpu.make_async_copy(bufs[b], _dst(b), sems[b]).wait()


_sc_call = functools.partial(
    pl.kernel,
    out_type=jax.ShapeDtypeStruct((_B, _T, _DEPTH), jnp.float32),
    mesh=plsc.VectorSubcoreMesh(core_axis_name="c", subcore_axis_name="s"),
    scratch_types=[
        pltpu.VMEM((_RPW + _LANE,), jnp.int32),
        pltpu.VMEM((_CH, _DEPTH), jnp.float32),
        pltpu.VMEM((_CH, _DEPTH), jnp.float32),
        pltpu.SemaphoreType.DMA,
        pltpu.SemaphoreType.DMA,
    ],
    compiler_params=pltpu.CompilerParams(needs_layout_passes=False),
)(_sc_body)


def kernel(x):
    return _sc_call(jnp.reshape(x, (_B * _T,)))


# speedup vs baseline: 4.2245x; 4.2245x over previous
import jax
import jax.numpy as jnp
from jax.experimental import pallas as pl

_DEPTH = 1000
_B, _T = 1024, 200


def _onehot_body(x_ref, o_ref):
    xv = x_ref[0, 0, :] % _DEPTH
    d = jax.lax.broadcasted_iota(jnp.int32, (1, _DEPTH, _B), 1)
    o_ref[...] = (d == xv[None, None, :]).astype(jnp.float32)


def kernel(x):
    xt = jnp.reshape(jnp.transpose(x, (1, 0)), (_T, 1, _B))
    out = pl.pallas_call(
        _onehot_body,
        grid=(_T,),
        in_specs=[pl.BlockSpec((1, 1, _B), lambda i: (i, 0, 0))],
        out_specs=pl.BlockSpec((1, _DEPTH, _B), lambda i: (i, 0, 0)),
        out_shape=jax.ShapeDtypeStruct((_T, _DEPTH, _B), jnp.float32),
    )(xt)
    return jnp.transpose(out, (2, 0, 1))
